# Initial kernel scaffold; baseline (speedup 1.0000x reference)
import jax
import jax.numpy as jnp
from jax.experimental import pallas as pl


def kernel(x, table, W, b):
    raise NotImplementedError("write your pallas kernel here")


# speedup vs baseline: 2.4008x; 2.4008x over previous
import functools

import jax
import jax.numpy as jnp
from jax import lax
from jax.experimental import pallas as pl
from jax.experimental.pallas import tpu as pltpu
from jax.experimental.pallas import tpu_sc as plsc

B = 4096
S = 200
D = 32
C = 20

NC = 2
NS = 16
NW = NC * NS
BPW = B // NW
G = 4
CHUNK = 100
RPG = G * S // CHUNK
NG = BPW // G
INV_S = 1.0 / S


def _sc_pooled(x4, table):
    mesh = plsc.VectorSubcoreMesh(core_axis_name="c", subcore_axis_name="s")

    @functools.partial(
        pl.kernel,
        out_type=jax.ShapeDtypeStruct((B, D), jnp.float32),
        mesh=mesh,
        compiler_params=pltpu.CompilerParams(use_tc_tiling_on_sc=False),
        scratch_types=[
            pltpu.VMEM((NG, RPG, CHUNK), jnp.int32),
            pltpu.VMEM((RPG, CHUNK, D), jnp.float32),
            pltpu.VMEM((RPG, CHUNK, D), jnp.float32),
            pltpu.VMEM((BPW, D), jnp.float32),
            pltpu.SemaphoreType.DMA,
            pltpu.SemaphoreType.DMA,
        ],
    )
    def k(x_hbm, table_hbm, out_hbm, idx_v, buf_a, buf_b, out_v, sem_a, sem_b):
        wid = lax.axis_index("s") * NC + lax.axis_index("c")
        row0 = wid * BPW
        pltpu.sync_copy(x_hbm.at[wid], idx_v)

        def gather_start(g, buf, sem):
            for j in range(RPG):
                pltpu.make_async_copy(
                    table_hbm.at[idx_v.at[g, j]], buf.at[j], sem
                ).start()

        def gather_wait(g, buf, sem):
            for j in range(RPG):
                pltpu.make_async_copy(
                    table_hbm.at[idx_v.at[g, j]], buf.at[j], sem
                ).wait()

        def reduce_group(buf, orow0):
            def body(s, accs):
                accs = list(accs)
                for i in range(G):
                    for j in (2 * i, 2 * i + 1):
                        accs[2 * i] = accs[2 * i] + buf[j, s, pl.ds(0, 16)]
                        accs[2 * i + 1] = accs[2 * i + 1] + buf[j, s, pl.ds(16, 16)]
                return tuple(accs)

            zero = jnp.zeros((16,), jnp.float32)
            accs = lax.fori_loop(0, CHUNK, body, (zero,) * (2 * G))
            for i in range(G):
                out_v[orow0 + i, pl.ds(0, 16)] = accs[2 * i] * INV_S
                out_v[orow0 + i, pl.ds(16, 16)] = accs[2 * i + 1] * INV_S

        gather_start(0, buf_a, sem_a)
        gather_start(1, buf_b, sem_b)

        def outer(gp, carry):
            g = 2 * gp
            gather_wait(g, buf_a, sem_a)
            reduce_group(buf_a, g * G)

            @pl.when(g + 2 < NG)
            def _():
                gather_start(g + 2, buf_a, sem_a)

            gather_wait(g + 1, buf_b, sem_b)
            reduce_group(buf_b, (g + 1) * G)

            @pl.when(g + 3 < NG)
            def _():
                gather_start(g + 3, buf_b, sem_b)

            return carry

        lax.fori_loop(0, NG // 2, outer, 0)
        pltpu.sync_copy(out_v, out_hbm.at[pl.ds(row0, BPW)])

    return k(x4, table)


def _head_body(p_ref, wt_ref, b_ref, o_ref):
    o_ref[...] = (
        jnp.dot(p_ref[...], wt_ref[...], preferred_element_type=jnp.float32)
        + b_ref[...]
    )


def _tc_head(pooled, Wt, b2):
    return pl.pallas_call(
        _head_body,
        out_shape=jax.ShapeDtypeStruct((B, C), jnp.float32),
    )(pooled, Wt, b2)


def kernel(x, table, W, b):
    x4 = x.astype(jnp.int32).reshape(NW, NG, RPG, CHUNK)
    pooled = _sc_pooled(x4, table)
    return _tc_head(pooled, W.T, b.reshape(1, C))
